# baseline (speedup 1.0000x reference)
import functools

import jax
import jax.numpy as jnp
from jax import lax
from jax.experimental import pallas as pl
from jax.experimental.pallas import tpu as pltpu
from jax.experimental.pallas import tpu_sc as plsc

_T = 32768
_E = 8
_BLK = 512
_BLK_SHIFT = 9
_XS = _T + _E * _BLK
_NB = _XS // _BLK
_NBP = 48
_DI = 768
_DH = 384
_DO = 256


def _pack_params(params):
    eye = jnp.eye(_DH, dtype=jnp.float32)
    zb = jnp.zeros((_DH,), jnp.float32)
    shapes = [(_DI, _DH), (_DH, _DH), (_DH, _DH), (_DH, _DH), (_DH, _DO)]
    layers = [[] for _ in range(5)]
    biases = [[] for _ in range(5)]
    for mlp in params:
        d = len(mlp)
        if d == 3:
            seq = [mlp[0], mlp[1], None, None, mlp[2]]
        elif d == 4:
            seq = [mlp[0], mlp[1], mlp[2], None, mlp[3]]
        else:
            seq = list(mlp)
        for i, (sh, wb) in enumerate(zip(shapes, seq)):
            if wb is None:
                W, b = eye, zb
            else:
                W, b = wb
                W = jnp.pad(W, ((0, sh[0] - W.shape[0]), (0, sh[1] - W.shape[1])))
                b = jnp.pad(b, (0, sh[1] - b.shape[0]))
            layers[i].append(W)
            biases[i].append(b)
    Ws = [jnp.stack(layers[i]).astype(jnp.bfloat16) for i in range(5)]
    Bs = [jnp.stack(biases[i]).reshape(_E, 1, -1) for i in range(5)]
    return (*Ws, *Bs)


def _psum_incl(x):
    io = lax.iota(jnp.int32, 16)
    for k in (1, 2, 4, 8):
        idx = jnp.maximum(io - k, 0)
        sh = x.at[idx].get(mode="promise_in_bounds")
        ge = jnp.minimum(jnp.maximum(io - (k - 1), 0), 1)
        x = x + sh * ge
    return x


def _splat_last(x):
    return x.at[jnp.full((16,), 15, jnp.int32)].get(mode="promise_in_bounds")


def _eq_mask(v, e):
    return 1 - jnp.minimum(jnp.abs(v - e), 1)


def _route_fn(plf_hbm, dest_hbm, be_hbm, pl_v, dest_v, cnt_v, all_v, be_v,
              shared_cnt, T=None):
    wid = lax.axis_index("s")
    C = T // 16
    G = C // 128
    zero = jnp.zeros((16,), jnp.int32)

    pltpu.sync_copy(plf_hbm.at[pl.ds(wid * C, C)], pl_v)

    def p1_body(i, accs):
        v = pl_v[pl.ds(i * 16, 16)]
        v = jnp.minimum(jnp.maximum(v, 0), _E - 1)
        return tuple(accs[e] + _eq_mask(v, e) for e in range(_E))

    accs = lax.fori_loop(0, C // 16, p1_body, tuple(zero for _ in range(_E)))
    for e in range(_E):
        cnt_v[pl.ds(e * 16, 16)] = _splat_last(_psum_incl(accs[e]))
    pltpu.sync_copy(cnt_v, shared_cnt.at[pl.ds(wid * _E * 16, _E * 16)])
    plsc.subcore_barrier()
    pltpu.sync_copy(shared_cnt, all_v)

    widv = jnp.broadcast_to(wid, (16,)).astype(jnp.int32)
    tot = []
    pref = []
    for e in range(_E):
        t = zero
        p = zero
        for w in range(16):
            c = all_v[pl.ds((w * _E + e) * 16, 16)]
            lt = jnp.minimum(jnp.maximum(widv - w, 0), 1)
            t = t + c
            p = p + c * lt
        tot.append(t)
        pref.append(p)
    off = [zero]
    for e in range(_E):
        off.append(off[e] + lax.shift_left(
            lax.shift_right_logical(tot[e] + (_BLK - 1), _BLK_SHIFT),
            _BLK_SHIFT))
    start = [off[e] + pref[e] for e in range(_E)]

    rs = tuple(start)
    for g in range(G):
        def p3_body(k, rs, g=g):
            v = pl_v[pl.ds((g * 8 + k) * 16, 16)]
            v = jnp.minimum(jnp.maximum(v, 0), _E - 1)
            d = zero
            rs = list(rs)
            for e in range(_E):
                eq = _eq_mask(v, e)
                incl = _psum_incl(eq)
                d = d + eq * (rs[e] + incl - eq)
                rs[e] = rs[e] + _splat_last(incl)
            dest_v[pl.ds((g * 8 + k) * 16, 16)] = d
            return tuple(rs)

        rs = lax.fori_loop(0, 8, p3_body, rs)
    for g in range(G):
        pltpu.sync_copy(dest_v.at[pl.ds(g * 128, 128)],
                        dest_hbm.at[wid * G + g])

    @pl.when(wid == 0)
    def _():
        for j in range(_NBP // 16):
            bs = (lax.iota(jnp.int32, 16) + j * 16) * _BLK
            acc = zero
            for e in range(1, _E + 1):
                acc = acc + jnp.minimum(jnp.maximum(bs - off[e] + 1, 0), 1)
            be_v[pl.ds(j * 16, 16)] = jnp.minimum(acc, _E - 1)
        pltpu.sync_copy(be_v, be_hbm)


def _route(plf, T):
    mesh = plsc.VectorSubcoreMesh(core_axis_name="c", subcore_axis_name="s",
                                  num_cores=1)
    f = pl.kernel(
        functools.partial(_route_fn, T=T),
        mesh=mesh,
        out_type=[jax.ShapeDtypeStruct((T // 128, 128), jnp.int32),
                  jax.ShapeDtypeStruct((_NBP,), jnp.int32)],
        scratch_types=[pltpu.VMEM((T // 16,), jnp.int32),
                       pltpu.VMEM((T // 16,), jnp.int32),
                       pltpu.VMEM((_E * 16,), jnp.int32),
                       pltpu.VMEM((16 * _E * 16,), jnp.int32),
                       pltpu.VMEM((_NBP,), jnp.int32),
                       pltpu.VMEM_SHARED((16 * _E * 16,), jnp.int32)],
    )
    return f(plf)


def _scatter_fn(x_hbm, dest_hbm, xs_hbm, didx_v, idx0_v, idx1_v, ra_v, rb_v,
                sr0, sr1, sw0, sw1, T=None):
    wid = lax.axis_index("s") * 2 + lax.axis_index("c")
    C = T // 32
    NG = C // 64
    pltpu.sync_copy(dest_hbm.at[pl.ds(wid * (C // 128), C // 128)], didx_v)
    idxs = (idx0_v, idx1_v)
    bufs = (ra_v, rb_v)
    rsem = (sr0, sr1)
    wsem = (sw0, sw1)

    def prep_idx(g, b):
        for j in range(4):
            idxs[b][pl.ds(j * 16, 16)] = didx_v[g // 2,
                                                pl.ds((g % 2) * 64 + j * 16, 16)]

    def read(g, b):
        return pltpu.async_copy(x_hbm.at[pl.ds(wid * C + g * 64, 64)],
                                bufs[b], rsem[b])

    def write(b):
        return pltpu.async_copy(bufs[b], xs_hbm.at[idxs[b]], wsem[b])

    prep_idx(0, 0)
    rh = {0: read(0, 0)}
    prep_idx(1, 1)
    rh[1] = read(1, 1)
    wh = {}
    for g in range(NG):
        b = g & 1
        rh[g].wait()
        wh[g] = write(b)
        if g + 2 < NG:
            wh[g].wait()
            prep_idx(g + 2, b)
            rh[g + 2] = read(g + 2, b)
    wh[NG - 2].wait()
    wh[NG - 1].wait()


def _scatter(x2, dest2, T, XS):
    mesh = plsc.VectorSubcoreMesh(core_axis_name="c", subcore_axis_name="s",
                                  num_cores=2)
    f = pl.kernel(
        functools.partial(_scatter_fn, T=T),
        mesh=mesh,
        out_type=[jax.ShapeDtypeStruct((XS, _DI), jnp.float32)],
        scratch_types=[pltpu.VMEM((T // 32 // 128, 128), jnp.int32),
                       pltpu.VMEM((64,), jnp.int32),
                       pltpu.VMEM((64,), jnp.int32),
                       pltpu.VMEM((64, _DI), jnp.float32),
                       pltpu.VMEM((64, _DI), jnp.float32),
                       pltpu.SemaphoreType.DMA,
                       pltpu.SemaphoreType.DMA,
                       pltpu.SemaphoreType.DMA,
                       pltpu.SemaphoreType.DMA],
    )
    return f(x2, dest2)[0]


def _gather_fn(ys_hbm, dest_hbm, out_hbm, didx_v, ra_v, rb_v,
               sr0, sr1, sw0, sw1, T=None):
    wid = lax.axis_index("s") * 2 + lax.axis_index("c")
    C = T // 32
    NG = C // 128
    pltpu.sync_copy(dest_hbm.at[pl.ds(wid * NG, NG)], didx_v)
    bufs = (ra_v, rb_v)
    rsem = (sr0, sr1)
    wsem = (sw0, sw1)

    def gat(g, b):
        return pltpu.async_copy(ys_hbm.at[didx_v.at[g]], bufs[b], rsem[b])

    def put(g, b):
        return pltpu.async_copy(bufs[b],
                                out_hbm.at[pl.ds(wid * C + g * 128, 128)],
                                wsem[b])

    rh = {0: gat(0, 0), 1: gat(1, 1)}
    wh = {}
    for g in range(NG):
        b = g & 1
        rh[g].wait()
        wh[g] = put(g, b)
        if g + 2 < NG:
            wh[g].wait()
            rh[g + 2] = gat(g + 2, b)
    wh[NG - 2].wait()
    wh[NG - 1].wait()


def _gather(ys, dest2, T):
    mesh = plsc.VectorSubcoreMesh(core_axis_name="c", subcore_axis_name="s",
                                  num_cores=2)
    f = pl.kernel(
        functools.partial(_gather_fn, T=T),
        mesh=mesh,
        out_type=[jax.ShapeDtypeStruct((T, _DO), jnp.int32 if False else jnp.float32)],
        scratch_types=[pltpu.VMEM((T // 32 // 128, 128), jnp.int32),
                       pltpu.VMEM((128, _DO), jnp.float32),
                       pltpu.VMEM((128, _DO), jnp.float32),
                       pltpu.SemaphoreType.DMA,
                       pltpu.SemaphoreType.DMA,
                       pltpu.SemaphoreType.DMA,
                       pltpu.SemaphoreType.DMA],
    )
    return f(ys, dest2)[0]


def _mlp_body(be_ref, x_ref, w0, w1, w2, w3, w4, b0, b1, b2, b3, b4, o_ref,
              h_ref):
    f32 = jnp.float32
    bf16 = jnp.bfloat16
    e = be_ref[pl.program_id(0)]
    h = jnp.dot(x_ref[...].astype(bf16), w0[0], preferred_element_type=f32)
    h = h + b0[0]
    h = jnp.maximum(h, 0.0).astype(bf16)
    h = jnp.dot(h, w1[0], preferred_element_type=f32) + b1[0]
    h_ref[...] = jnp.maximum(h, 0.0).astype(bf16)

    @pl.when(e >= 2)
    def _():
        t = jnp.dot(h_ref[...], w2[0], preferred_element_type=f32) + b2[0]
        h_ref[...] = jnp.maximum(t, 0.0).astype(bf16)

    @pl.when(e >= 4)
    def _():
        t = jnp.dot(h_ref[...], w3[0], preferred_element_type=f32) + b3[0]
        h_ref[...] = jnp.maximum(t, 0.0).astype(bf16)

    o_ref[...] = jnp.dot(h_ref[...], w4[0], preferred_element_type=f32) + b4[0]


def _mlp(be, xs, w0, w1, w2, w3, w4, b0, b1, b2, b3, b4, NB, XS):
    def wmap(i, be_ref):
        return (be_ref[i], 0, 0)

    grid_spec = pltpu.PrefetchScalarGridSpec(
        num_scalar_prefetch=1,
        grid=(NB,),
        in_specs=[
            pl.BlockSpec((_BLK, _DI), lambda i, be_ref: (i, 0)),
            pl.BlockSpec((1, _DI, _DH), wmap),
            pl.BlockSpec((1, _DH, _DH), wmap),
            pl.BlockSpec((1, _DH, _DH), wmap),
            pl.BlockSpec((1, _DH, _DH), wmap),
            pl.BlockSpec((1, _DH, _DO), wmap),
            pl.BlockSpec((1, 1, _DH), wmap),
            pl.BlockSpec((1, 1, _DH), wmap),
            pl.BlockSpec((1, 1, _DH), wmap),
            pl.BlockSpec((1, 1, _DH), wmap),
            pl.BlockSpec((1, 1, _DO), wmap),
        ],
        out_specs=pl.BlockSpec((_BLK, _DO), lambda i, be_ref: (i, 0)),
        scratch_shapes=[pltpu.VMEM((_BLK, _DH), jnp.bfloat16)],
    )
    return pl.pallas_call(
        _mlp_body,
        grid_spec=grid_spec,
        out_shape=jax.ShapeDtypeStruct((XS, _DO), jnp.float32),
    )(be, xs, w0, w1, w2, w3, w4, b0, b1, b2, b3, b4)


def kernel(x, path_lengths, params):
    B, N, _ = x.shape
    packed = _pack_params(params)
    CH = 2
    Tc = _T // CH
    XSc = Tc + _E * _BLK
    NBc = XSc // _BLK
    outs = []
    x3 = x.reshape(CH, Tc, _DI)
    pl3 = jnp.clip(path_lengths.reshape(CH, Tc), 0, _E - 1).astype(jnp.int32)
    for c in range(CH):
        dest2, be = _route(pl3[c], Tc)
        xs = _scatter(x3[c], dest2, Tc, XSc)
        ys = _mlp(be[:NBc], xs, *packed, NB=NBc, XS=XSc)
        outs.append(_gather(ys, dest2, Tc))
    return jnp.concatenate(outs, axis=0).reshape(B, N, _DO)


# speedup vs baseline: 1.3427x; 1.3427x over previous
import functools

import jax
import jax.numpy as jnp
from jax import lax
from jax.experimental import pallas as pl
from jax.experimental.pallas import tpu as pltpu
from jax.experimental.pallas import tpu_sc as plsc

_T = 32768
_E = 8
_BLK = 1024
_BLK_SHIFT = 10
_XS = _T + _E * _BLK
_NB = _XS // _BLK
_NBP = 48
_DI = 768
_DH = 384
_DO = 256


def _pack_params(params):
    eye = jnp.eye(_DH, dtype=jnp.float32)
    zb = jnp.zeros((_DH,), jnp.float32)
    shapes = [(_DI, _DH), (_DH, _DH), (_DH, _DH), (_DH, _DH), (_DH, _DO)]
    layers = [[] for _ in range(5)]
    biases = [[] for _ in range(5)]
    for mlp in params:
        d = len(mlp)
        if d == 3:
            seq = [mlp[0], mlp[1], None, None, mlp[2]]
        elif d == 4:
            seq = [mlp[0], mlp[1], mlp[2], None, mlp[3]]
        else:
            seq = list(mlp)
        for i, (sh, wb) in enumerate(zip(shapes, seq)):
            if wb is None:
                W, b = eye, zb
            else:
                W, b = wb
                W = jnp.pad(W, ((0, sh[0] - W.shape[0]), (0, sh[1] - W.shape[1])))
                b = jnp.pad(b, (0, sh[1] - b.shape[0]))
            layers[i].append(W)
            biases[i].append(b)
    Ws = [jnp.stack(layers[i]).astype(jnp.bfloat16) for i in range(5)]
    Bs = [jnp.stack(biases[i]).reshape(_E, 1, -1) for i in range(5)]
    return (*Ws, *Bs)


def _psum_incl(x):
    io = lax.iota(jnp.int32, 16)
    for k in (1, 2, 4, 8):
        idx = jnp.maximum(io - k, 0)
        sh = x.at[idx].get(mode="promise_in_bounds")
        ge = jnp.minimum(jnp.maximum(io - (k - 1), 0), 1)
        x = x + sh * ge
    return x


def _splat_last(x):
    return x.at[jnp.full((16,), 15, jnp.int32)].get(mode="promise_in_bounds")


def _eq_mask(v, e):
    return 1 - jnp.minimum(jnp.abs(v - e), 1)


def _route_fn(plf_hbm, dest_hbm, be_hbm, pl_v, dest_v, cnt_v, all_v, be_v,
              shared_cnt):
    wid = lax.axis_index("s")
    C = _T // 16
    G = C // 128
    zero = jnp.zeros((16,), jnp.int32)

    pltpu.sync_copy(plf_hbm.at[pl.ds(wid * C, C)], pl_v)

    def p1_body(i, accs):
        v = pl_v[pl.ds(i * 16, 16)]
        v = jnp.minimum(jnp.maximum(v, 0), _E - 1)
        return tuple(accs[e] + _eq_mask(v, e) for e in range(_E))

    accs = lax.fori_loop(0, C // 16, p1_body, tuple(zero for _ in range(_E)))
    for e in range(_E):
        cnt_v[pl.ds(e * 16, 16)] = _splat_last(_psum_incl(accs[e]))
    pltpu.sync_copy(cnt_v, shared_cnt.at[pl.ds(wid * _E * 16, _E * 16)])
    plsc.subcore_barrier()
    pltpu.sync_copy(shared_cnt, all_v)

    widv = jnp.broadcast_to(wid, (16,)).astype(jnp.int32)
    tot = []
    pref = []
    for e in range(_E):
        t = zero
        p = zero
        for w in range(16):
            c = all_v[pl.ds((w * _E + e) * 16, 16)]
            lt = jnp.minimum(jnp.maximum(widv - w, 0), 1)
            t = t + c
            p = p + c * lt
        tot.append(t)
        pref.append(p)
    off = [zero]
    for e in range(_E):
        off.append(off[e] + lax.shift_left(
            lax.shift_right_logical(tot[e] + (_BLK - 1), _BLK_SHIFT),
            _BLK_SHIFT))
    start = [off[e] + pref[e] for e in range(_E)]

    rs = tuple(start)
    for g in range(G):
        def p3_body(k, rs, g=g):
            v = pl_v[pl.ds((g * 8 + k) * 16, 16)]
            v = jnp.minimum(jnp.maximum(v, 0), _E - 1)
            d = zero
            rs = list(rs)
            for e in range(_E):
                eq = _eq_mask(v, e)
                incl = _psum_incl(eq)
                d = d + eq * (rs[e] + incl - eq)
                rs[e] = rs[e] + _splat_last(incl)
            dest_v[pl.ds((g * 8 + k) * 16, 16)] = d
            return tuple(rs)

        rs = lax.fori_loop(0, 8, p3_body, rs)
    for g in range(G):
        pltpu.sync_copy(dest_v.at[pl.ds(g * 128, 128)],
                        dest_hbm.at[wid * G + g])

    @pl.when(wid == 0)
    def _():
        for j in range(_NBP // 16):
            bs = (lax.iota(jnp.int32, 16) + j * 16) * _BLK
            acc = zero
            for e in range(1, _E + 1):
                acc = acc + jnp.minimum(jnp.maximum(bs - off[e] + 1, 0), 1)
            be_v[pl.ds(j * 16, 16)] = jnp.minimum(acc, _E - 1)
        pltpu.sync_copy(be_v, be_hbm)


def _route(plf):
    mesh = plsc.VectorSubcoreMesh(core_axis_name="c", subcore_axis_name="s",
                                  num_cores=1)
    f = pl.kernel(
        _route_fn,
        mesh=mesh,
        out_type=[jax.ShapeDtypeStruct((_T // 128, 128), jnp.int32),
                  jax.ShapeDtypeStruct((_NBP,), jnp.int32)],
        scratch_types=[pltpu.VMEM((_T // 16,), jnp.int32),
                       pltpu.VMEM((_T // 16,), jnp.int32),
                       pltpu.VMEM((_E * 16,), jnp.int32),
                       pltpu.VMEM((16 * _E * 16,), jnp.int32),
                       pltpu.VMEM((_NBP,), jnp.int32),
                       pltpu.VMEM_SHARED((16 * _E * 16,), jnp.int32)],
    )
    return f(plf)


def _scatter_fn(x_hbm, dest_hbm, xs_hbm, didx_v, idx0_v, idx1_v, ra_v, rb_v,
                sr0, sr1, sw0, sw1):
    wid = lax.axis_index("s") * 2 + lax.axis_index("c")
    C = _T // 32
    NG = C // 64
    pltpu.sync_copy(dest_hbm.at[pl.ds(wid * (C // 128), C // 128)], didx_v)
    idxs = (idx0_v, idx1_v)
    bufs = (ra_v, rb_v)
    rsem = (sr0, sr1)
    wsem = (sw0, sw1)

    def prep_idx(g, b):
        for j in range(4):
            idxs[b][pl.ds(j * 16, 16)] = didx_v[g // 2,
                                                pl.ds((g % 2) * 64 + j * 16, 16)]

    def read(g, b):
        return pltpu.async_copy(x_hbm.at[pl.ds(wid * C + g * 64, 64)],
                                bufs[b], rsem[b])

    def write(b):
        return pltpu.async_copy(bufs[b], xs_hbm.at[idxs[b]], wsem[b])

    prep_idx(0, 0)
    rh = {0: read(0, 0)}
    prep_idx(1, 1)
    rh[1] = read(1, 1)
    wh = {}
    for g in range(NG):
        b = g & 1
        rh[g].wait()
        wh[g] = write(b)
        if g + 2 < NG:
            wh[g].wait()
            prep_idx(g + 2, b)
            rh[g + 2] = read(g + 2, b)
    wh[NG - 2].wait()
    wh[NG - 1].wait()


def _scatter(x2, dest2):
    mesh = plsc.VectorSubcoreMesh(core_axis_name="c", subcore_axis_name="s",
                                  num_cores=2)
    f = pl.kernel(
        _scatter_fn,
        mesh=mesh,
        out_type=[jax.ShapeDtypeStruct((_XS, _DI), jnp.float32)],
        scratch_types=[pltpu.VMEM((_T // 32 // 128, 128), jnp.int32),
                       pltpu.VMEM((64,), jnp.int32),
                       pltpu.VMEM((64,), jnp.int32),
                       pltpu.VMEM((64, _DI), jnp.float32),
                       pltpu.VMEM((64, _DI), jnp.float32),
                       pltpu.SemaphoreType.DMA,
                       pltpu.SemaphoreType.DMA,
                       pltpu.SemaphoreType.DMA,
                       pltpu.SemaphoreType.DMA],
    )
    return f(x2, dest2)[0]


def _gather_fn(ys_hbm, dest_hbm, out_hbm, didx_v, ra_v, rb_v,
               sr0, sr1, sw0, sw1):
    wid = lax.axis_index("s") * 2 + lax.axis_index("c")
    C = _T // 32
    NG = C // 128
    pltpu.sync_copy(dest_hbm.at[pl.ds(wid * NG, NG)], didx_v)
    bufs = (ra_v, rb_v)
    rsem = (sr0, sr1)
    wsem = (sw0, sw1)

    def gat(g, b):
        return pltpu.async_copy(ys_hbm.at[didx_v.at[g]], bufs[b], rsem[b])

    def put(g, b):
        return pltpu.async_copy(bufs[b],
                                out_hbm.at[pl.ds(wid * C + g * 128, 128)],
                                wsem[b])

    rh = {0: gat(0, 0), 1: gat(1, 1)}
    wh = {}
    for g in range(NG):
        b = g & 1
        rh[g].wait()
        wh[g] = put(g, b)
        if g + 2 < NG:
            wh[g].wait()
            rh[g + 2] = gat(g + 2, b)
    wh[NG - 2].wait()
    wh[NG - 1].wait()


def _gather(ys, dest2):
    mesh = plsc.VectorSubcoreMesh(core_axis_name="c", subcore_axis_name="s",
                                  num_cores=2)
    f = pl.kernel(
        _gather_fn,
        mesh=mesh,
        out_type=[jax.ShapeDtypeStruct((_T, _DO), jnp.float32)],
        scratch_types=[pltpu.VMEM((_T // 32 // 128, 128), jnp.int32),
                       pltpu.VMEM((128, _DO), jnp.float32),
                       pltpu.VMEM((128, _DO), jnp.float32),
                       pltpu.SemaphoreType.DMA,
                       pltpu.SemaphoreType.DMA,
                       pltpu.SemaphoreType.DMA,
                       pltpu.SemaphoreType.DMA],
    )
    return f(ys, dest2)[0]


def _mlp_body(be_ref, x_ref, w0, w1, w2, w3, w4, b0, b1, b2, b3, b4, o_ref,
              h_ref):
    f32 = jnp.float32
    bf16 = jnp.bfloat16
    e = be_ref[pl.program_id(0)]
    h = jnp.dot(x_ref[...].astype(bf16), w0[0], preferred_element_type=f32)
    h = h + b0[0]
    h = jnp.maximum(h, 0.0).astype(bf16)
    h = jnp.dot(h, w1[0], preferred_element_type=f32) + b1[0]
    h_ref[...] = jnp.maximum(h, 0.0).astype(bf16)

    @pl.when(e >= 2)
    def _():
        t = jnp.dot(h_ref[...], w2[0], preferred_element_type=f32) + b2[0]
        h_ref[...] = jnp.maximum(t, 0.0).astype(bf16)

    @pl.when(e >= 4)
    def _():
        t = jnp.dot(h_ref[...], w3[0], preferred_element_type=f32) + b3[0]
        h_ref[...] = jnp.maximum(t, 0.0).astype(bf16)

    o_ref[...] = jnp.dot(h_ref[...], w4[0], preferred_element_type=f32) + b4[0]


def _mlp(be, xs, w0, w1, w2, w3, w4, b0, b1, b2, b3, b4):
    def wmap(i, be_ref):
        return (be_ref[i], 0, 0)

    grid_spec = pltpu.PrefetchScalarGridSpec(
        num_scalar_prefetch=1,
        grid=(_NB,),
        in_specs=[
            pl.BlockSpec((_BLK, _DI), lambda i, be_ref: (i, 0)),
            pl.BlockSpec((1, _DI, _DH), wmap),
            pl.BlockSpec((1, _DH, _DH), wmap),
            pl.BlockSpec((1, _DH, _DH), wmap),
            pl.BlockSpec((1, _DH, _DH), wmap),
            pl.BlockSpec((1, _DH, _DO), wmap),
            pl.BlockSpec((1, 1, _DH), wmap),
            pl.BlockSpec((1, 1, _DH), wmap),
            pl.BlockSpec((1, 1, _DH), wmap),
            pl.BlockSpec((1, 1, _DH), wmap),
            pl.BlockSpec((1, 1, _DO), wmap),
        ],
        out_specs=pl.BlockSpec((_BLK, _DO), lambda i, be_ref: (i, 0)),
        scratch_shapes=[pltpu.VMEM((_BLK, _DH), jnp.bfloat16)],
    )
    return pl.pallas_call(
        _mlp_body,
        grid_spec=grid_spec,
        out_shape=jax.ShapeDtypeStruct((_XS, _DO), jnp.float32),
    )(be, xs, w0, w1, w2, w3, w4, b0, b1, b2, b3, b4)


def kernel(x, path_lengths, params):
    B, N, _ = x.shape
    x2 = x.reshape(_T, _DI)
    plf = jnp.clip(path_lengths.reshape(_T), 0, _E - 1).astype(jnp.int32)
    packed = _pack_params(params)
    dest2, be = _route(plf)
    xs = _scatter(x2, dest2)
    ys = _mlp(be[:_NB], xs, *packed)
    out2 = _gather(ys, dest2)
    return out2.reshape(B, N, _DO)
